# Initial kernel scaffold; baseline (speedup 1.0000x reference)
import jax
import jax.numpy as jnp
from jax.experimental import pallas as pl


def kernel(features, edge_index, W_in, b_in, W_self1, W_neigh1, b1, W_self2, W_neigh2, b2, W_out, b_out):
    raise NotImplementedError("write your pallas kernel here")


# speedup vs baseline: 5.4527x; 5.4527x over previous
import jax
import jax.numpy as jnp
from jax import lax
from jax.experimental import pallas as pl
from jax.experimental.pallas import tpu as pltpu
from jax.experimental.pallas import tpu_sc as plsc

NEG_SLOPE = 0.01
NC = 2
NS = 16
WIN = 128


def _seg_sum_sc(n, d, e, count_only=False):
    nwin = e // WIN
    assert nwin * WIN == e and nwin % NC == 0
    win_per_core = nwin // NC
    rows_per_sub = (n // NS) // 8 * 8
    tail = n - rows_per_sub * NS
    assert rows_per_sub % 8 == 0 and tail % 8 == 0 and 0 < WIN
    nfull = rows_per_sub // WIN
    rem = rows_per_sub % WIN

    mesh = plsc.VectorSubcoreMesh(core_axis_name="c", subcore_axis_name="s")
    out_type = jax.ShapeDtypeStruct((NC, n, d), jnp.float32)
    scratch = [
        pltpu.VMEM((WIN,), jnp.int32),
        pltpu.VMEM((WIN,), jnp.int32),
        pltpu.VMEM((WIN, d), jnp.float32),
        pltpu.VMEM_SHARED((n, d), jnp.float32),
        pltpu.SemaphoreType.DMA,
    ]

    def body(h_hbm, src_hbm, dst_hbm, out_hbm, src_v, dst_v, rows_v, acc_sh,
             sem):
        c = lax.axis_index("c")
        s = lax.axis_index("s")

        @pl.loop(0, WIN)
        def _(r):
            @pl.loop(0, d, step=16)
            def _(k):
                rows_v[r, pl.ds(k, 16)] = jnp.zeros((16,), jnp.float32)

        base_row = s * rows_per_sub
        is_last = s == NS - 1
        zoffs = [k * WIN for k in range(nfull)]
        if rem:
            zoffs.append(rows_per_sub - WIN)
        for off in zoffs:
            pltpu.sync_copy(rows_v, acc_sh.at[pl.ds(base_row + off, WIN)])
        if tail:
            @pl.when(is_last)
            def _():
                pltpu.sync_copy(rows_v, acc_sh.at[pl.ds(n - WIN, WIN)])

        if count_only:
            @pl.loop(0, WIN)
            def _(r):
                @pl.loop(0, d, step=16)
                def _(k):
                    rows_v[r, pl.ds(k, 16)] = jnp.ones((16,), jnp.float32)

        plsc.subcore_barrier()

        my_wins = (win_per_core - s + NS - 1) // NS

        @pl.loop(0, my_wins)
        def _(t):
            win = c * win_per_core + s + t * NS
            ebase = win * WIN
            pltpu.sync_copy(dst_hbm.at[pl.ds(ebase, WIN)], dst_v)
            if not count_only:
                pltpu.sync_copy(src_hbm.at[pl.ds(ebase, WIN)], src_v)
                pltpu.async_copy(h_hbm.at[src_v], rows_v, sem).wait()
            pltpu.sync_copy(rows_v, acc_sh.at[dst_v], add=True)

        plsc.subcore_barrier()

        pltpu.sync_copy(acc_sh.at[pl.ds(base_row, rows_per_sub)],
                        out_hbm.at[c, pl.ds(base_row, rows_per_sub)])
        if tail:
            @pl.when(is_last)
            def _():
                pltpu.sync_copy(acc_sh.at[pl.ds(n - tail, tail)],
                                out_hbm.at[c, pl.ds(n - tail, tail)])

    return pl.kernel(body, out_type=out_type, mesh=mesh, scratch_types=scratch)


def _dot(a, b):
    return lax.dot_general(a, b, (((1,), (0,)), ((), ())),
                           precision=lax.Precision.HIGHEST,
                           preferred_element_type=jnp.float32)


def _matmul_bias(x, w, b, blk=1000):
    n, k = x.shape
    m = w.shape[1]
    assert n % blk == 0

    def body(x_ref, w_ref, b_ref, o_ref):
        o_ref[...] = _dot(x_ref[...], w_ref[...]) + b_ref[...]

    return pl.pallas_call(
        body,
        grid=(n // blk,),
        in_specs=[
            pl.BlockSpec((blk, k), lambda i: (i, 0)),
            pl.BlockSpec((k, m), lambda i: (0, 0)),
            pl.BlockSpec((1, m), lambda i: (0, 0)),
        ],
        out_specs=pl.BlockSpec((blk, m), lambda i: (i, 0)),
        out_shape=jax.ShapeDtypeStruct((n, m), jnp.float32),
    )(x, w, b)


def _sage_combine(h, part, degp, w_self, w_neigh, b, w_out=None, b_out=None,
                  blk=1000):
    n, d = h.shape
    fuse = w_out is not None
    m = w_out.shape[1] if fuse else d
    assert n % blk == 0

    def body(h_ref, p_ref, dg_ref, ws_ref, wn_ref, b_ref, *rest):
        if fuse:
            wo_ref, bo_ref, o_ref = rest
        else:
            (o_ref,) = rest
        agg = p_ref[0] + p_ref[1]
        deg = dg_ref[0, :, 0:1] + dg_ref[1, :, 0:1]
        mean = agg / jnp.maximum(deg, 1.0)
        z = _dot(h_ref[...], ws_ref[...]) + _dot(mean, wn_ref[...]) + b_ref[...]
        z = jnp.where(z >= 0, z, NEG_SLOPE * z)
        if fuse:
            z = _dot(z, wo_ref[...]) + bo_ref[...]
        o_ref[...] = z

    in_specs = [
        pl.BlockSpec((blk, d), lambda i: (i, 0)),
        pl.BlockSpec((NC, blk, d), lambda i: (0, i, 0)),
        pl.BlockSpec((NC, blk, d), lambda i: (0, i, 0)),
        pl.BlockSpec((d, d), lambda i: (0, 0)),
        pl.BlockSpec((d, d), lambda i: (0, 0)),
        pl.BlockSpec((1, d), lambda i: (0, 0)),
    ]
    args = [h, part, degp, w_self, w_neigh, b]
    if fuse:
        in_specs.append(pl.BlockSpec((d, m), lambda i: (0, 0)))
        in_specs.append(pl.BlockSpec((1, m), lambda i: (0, 0)))
        args.extend([w_out, b_out])

    return pl.pallas_call(
        body,
        grid=(n // blk,),
        in_specs=in_specs,
        out_specs=pl.BlockSpec((blk, m), lambda i: (i, 0)),
        out_shape=jax.ShapeDtypeStruct((n, m), jnp.float32),
    )(*args)


def kernel(features, edge_index, W_in, b_in, W_self1, W_neigh1, b1,
           W_self2, W_neigh2, b2, W_out, b_out):
    n, d = features.shape
    e = edge_index.shape[1]
    src = edge_index[0]
    dst = edge_index[1]

    h0 = _matmul_bias(features, W_in, b_in.reshape(1, -1))
    seg = _seg_sum_sc(n, d, e)
    degp = _seg_sum_sc(n, d, e, count_only=True)(features, src, dst)
    part1 = seg(h0, src, dst)
    h1 = _sage_combine(h0, part1, degp, W_self1, W_neigh1, b1.reshape(1, -1))
    part2 = seg(h1, src, dst)
    out = _sage_combine(h1, part2, degp, W_self2, W_neigh2, b2.reshape(1, -1),
                        W_out, b_out.reshape(1, -1))
    return out


# speedup vs baseline: 9.4853x; 1.7396x over previous
import jax
import jax.numpy as jnp
from jax import lax
from jax.experimental import pallas as pl
from jax.experimental.pallas import tpu as pltpu
from jax.experimental.pallas import tpu_sc as plsc

NEG_SLOPE = 0.01
NC = 2
NS = 16
WIN = 128


def _seg_sum_sc(n, d, e, count_only=False):
    nwin = e // WIN
    assert nwin * WIN == e
    NW = NC * NS
    chunk = (-(-nwin // NW) + 7) // 8 * 8
    tail_chunk = nwin - chunk * (NW - 1)
    assert 0 < tail_chunk <= chunk and tail_chunk % 2 == 0 and chunk % 2 == 0
    rows_per_sub = (n // NS) // 8 * 8
    tail_rows = n - rows_per_sub * NS
    assert rows_per_sub % 8 == 0 and tail_rows % 8 == 0
    nfull = rows_per_sub // WIN
    rem = rows_per_sub % WIN

    mesh = plsc.VectorSubcoreMesh(core_axis_name="c", subcore_axis_name="s")
    out_type = jax.ShapeDtypeStruct((NC, n, d), jnp.float32)
    half = chunk // 2
    assert half % 2 == 0 and tail_chunk <= half
    scratch = [
        pltpu.VMEM((half, WIN), jnp.int32),
        pltpu.VMEM((half, WIN), jnp.int32),
        pltpu.VMEM((WIN, d), jnp.float32),
        pltpu.VMEM((WIN, d), jnp.float32),
        pltpu.VMEM_SHARED((n, d), jnp.float32),
        pltpu.SemaphoreType.DMA,
        pltpu.SemaphoreType.DMA,
    ]

    def body(h_hbm, src_hbm, dst_hbm, out_hbm, src_all, dst_all, rows0, rows1,
             acc_sh, sem0, sem1):
        c = lax.axis_index("c")
        s = lax.axis_index("s")
        k = c * NS + s

        @pl.loop(0, WIN)
        def _(r):
            @pl.loop(0, d, step=16)
            def _(q):
                rows0[r, pl.ds(q, 16)] = jnp.zeros((16,), jnp.float32)

        base_row = s * rows_per_sub
        is_last = s == NS - 1
        zoffs = [q * WIN for q in range(nfull)]
        if rem:
            zoffs.append(rows_per_sub - WIN)
        for off in zoffs:
            pltpu.sync_copy(rows0, acc_sh.at[pl.ds(base_row + off, WIN)])
        if tail_rows:
            @pl.when(is_last)
            def _():
                pltpu.sync_copy(rows0, acc_sh.at[pl.ds(n - WIN, WIN)])

        if count_only:
            @pl.loop(0, WIN)
            def _(r):
                @pl.loop(0, d, step=16)
                def _(q):
                    rows0[r, pl.ds(q, 16)] = jnp.ones((16,), jnp.float32)

        plsc.subcore_barrier()

        def run_chunk(m, wbase):
            pltpu.sync_copy(dst_hbm.at[pl.ds(wbase, m)],
                            dst_all.at[pl.ds(0, m)])
            if count_only:
                @pl.loop(0, m)
                def _(t):
                    pltpu.sync_copy(rows0, acc_sh.at[dst_all.at[t]], add=True)
                return
            pltpu.sync_copy(src_hbm.at[pl.ds(wbase, m)],
                            src_all.at[pl.ds(0, m)])
            pltpu.async_copy(h_hbm.at[src_all.at[0]], rows0, sem0)
            pltpu.async_copy(h_hbm.at[src_all.at[1]], rows1, sem1)

            @pl.loop(0, m // 2)
            def _(u):
                t0 = 2 * u
                pltpu.make_async_copy(h_hbm.at[src_all.at[t0]], rows0,
                                      sem0).wait()
                pltpu.sync_copy(rows0, acc_sh.at[dst_all.at[t0]], add=True)

                @pl.when(t0 + 2 < m)
                def _():
                    pltpu.async_copy(h_hbm.at[src_all.at[t0 + 2]], rows0, sem0)

                pltpu.make_async_copy(h_hbm.at[src_all.at[t0 + 1]], rows1,
                                      sem1).wait()
                pltpu.sync_copy(rows1, acc_sh.at[dst_all.at[t0 + 1]], add=True)

                @pl.when(t0 + 3 < m)
                def _():
                    pltpu.async_copy(h_hbm.at[src_all.at[t0 + 3]], rows1, sem1)

        @pl.when(k < NW - 1)
        def _():
            run_chunk(half, k * chunk)
            run_chunk(half, k * chunk + half)

        @pl.when(k == NW - 1)
        def _():
            run_chunk(tail_chunk, (NW - 1) * chunk)

        plsc.subcore_barrier()

        pltpu.sync_copy(acc_sh.at[pl.ds(base_row, rows_per_sub)],
                        out_hbm.at[c, pl.ds(base_row, rows_per_sub)])
        if tail_rows:
            @pl.when(is_last)
            def _():
                pltpu.sync_copy(acc_sh.at[pl.ds(n - tail_rows, tail_rows)],
                                out_hbm.at[c, pl.ds(n - tail_rows, tail_rows)])

    return pl.kernel(body, out_type=out_type, mesh=mesh, scratch_types=scratch)


def _dot(a, b):
    return lax.dot_general(a, b, (((1,), (0,)), ((), ())),
                           precision=lax.Precision.HIGHEST,
                           preferred_element_type=jnp.float32)


def _matmul_bias(x, w, b, blk=1000):
    n, k = x.shape
    m = w.shape[1]
    assert n % blk == 0

    def body(x_ref, w_ref, b_ref, o_ref):
        o_ref[...] = _dot(x_ref[...], w_ref[...]) + b_ref[...]

    return pl.pallas_call(
        body,
        grid=(n // blk,),
        in_specs=[
            pl.BlockSpec((blk, k), lambda i: (i, 0)),
            pl.BlockSpec((k, m), lambda i: (0, 0)),
            pl.BlockSpec((1, m), lambda i: (0, 0)),
        ],
        out_specs=pl.BlockSpec((blk, m), lambda i: (i, 0)),
        out_shape=jax.ShapeDtypeStruct((n, m), jnp.float32),
    )(x, w, b)


def _sage_combine(h, part, degp, w_self, w_neigh, b, w_out=None, b_out=None,
                  blk=1000):
    n, d = h.shape
    fuse = w_out is not None
    m = w_out.shape[1] if fuse else d
    assert n % blk == 0

    def body(h_ref, p_ref, dg_ref, ws_ref, wn_ref, b_ref, *rest):
        if fuse:
            wo_ref, bo_ref, o_ref = rest
        else:
            (o_ref,) = rest
        agg = p_ref[0] + p_ref[1]
        deg = dg_ref[0, :, 0:1] + dg_ref[1, :, 0:1]
        mean = agg / jnp.maximum(deg, 1.0)
        z = _dot(h_ref[...], ws_ref[...]) + _dot(mean, wn_ref[...]) + b_ref[...]
        z = jnp.where(z >= 0, z, NEG_SLOPE * z)
        if fuse:
            z = _dot(z, wo_ref[...]) + bo_ref[...]
        o_ref[...] = z

    in_specs = [
        pl.BlockSpec((blk, d), lambda i: (i, 0)),
        pl.BlockSpec((NC, blk, d), lambda i: (0, i, 0)),
        pl.BlockSpec((NC, blk, d), lambda i: (0, i, 0)),
        pl.BlockSpec((d, d), lambda i: (0, 0)),
        pl.BlockSpec((d, d), lambda i: (0, 0)),
        pl.BlockSpec((1, d), lambda i: (0, 0)),
    ]
    args = [h, part, degp, w_self, w_neigh, b]
    if fuse:
        in_specs.append(pl.BlockSpec((d, m), lambda i: (0, 0)))
        in_specs.append(pl.BlockSpec((1, m), lambda i: (0, 0)))
        args.extend([w_out, b_out])

    return pl.pallas_call(
        body,
        grid=(n // blk,),
        in_specs=in_specs,
        out_specs=pl.BlockSpec((blk, m), lambda i: (i, 0)),
        out_shape=jax.ShapeDtypeStruct((n, m), jnp.float32),
    )(*args)


def kernel(features, edge_index, W_in, b_in, W_self1, W_neigh1, b1,
           W_self2, W_neigh2, b2, W_out, b_out):
    n, d = features.shape
    e = edge_index.shape[1]
    src = edge_index[0].reshape(e // WIN, WIN)
    dst = edge_index[1].reshape(e // WIN, WIN)

    h0 = _matmul_bias(features, W_in, b_in.reshape(1, -1))
    seg = _seg_sum_sc(n, d, e)
    degp = _seg_sum_sc(n, d, e, count_only=True)(features, src, dst)
    part1 = seg(h0, src, dst)
    h1 = _sage_combine(h0, part1, degp, W_self1, W_neigh1, b1.reshape(1, -1))
    part2 = seg(h1, src, dst)
    out = _sage_combine(h1, part2, degp, W_self2, W_neigh2, b2.reshape(1, -1),
                        W_out, b_out.reshape(1, -1))
    return out


# speedup vs baseline: 10.7185x; 1.1300x over previous
import jax
import jax.numpy as jnp
from jax import lax
from jax.experimental import pallas as pl
from jax.experimental.pallas import tpu as pltpu
from jax.experimental.pallas import tpu_sc as plsc

NEG_SLOPE = 0.01
NC = 2
NS = 16
WIN = 128


def _seg_sum_sc(n, d, e, count_only=False):
    nwin = e // WIN
    assert nwin * WIN == e
    NW = NC * NS
    chunk = (-(-nwin // NW) + 7) // 8 * 8
    tail_chunk = nwin - chunk * (NW - 1)
    assert 0 < tail_chunk <= chunk and tail_chunk % 2 == 0 and chunk % 2 == 0
    rows_per_sub = (n // NS) // 8 * 8
    tail_rows = n - rows_per_sub * NS
    assert rows_per_sub % 8 == 0 and tail_rows % 8 == 0
    nfull = rows_per_sub // WIN
    rem = rows_per_sub % WIN

    mesh = plsc.VectorSubcoreMesh(core_axis_name="c", subcore_axis_name="s")
    out_type = jax.ShapeDtypeStruct((NC, n, d), jnp.float32)
    half = chunk // 2
    assert half % 2 == 0 and tail_chunk <= half
    scratch = [
        pltpu.VMEM((half, WIN), jnp.int32),
        pltpu.VMEM((half, WIN), jnp.int32),
        pltpu.VMEM((WIN, d), jnp.float32),
        pltpu.VMEM((WIN, d), jnp.float32),
        pltpu.VMEM_SHARED((n, d), jnp.float32),
        pltpu.SemaphoreType.DMA,
        pltpu.SemaphoreType.DMA,
    ]

    def body(h_hbm, ei_hbm, out_hbm, src_all, dst_all, rows0, rows1,
             acc_sh, sem0, sem1):
        c = lax.axis_index("c")
        s = lax.axis_index("s")
        k = c * NS + s

        @pl.loop(0, WIN)
        def _(r):
            @pl.loop(0, d, step=16)
            def _(q):
                rows0[r, pl.ds(q, 16)] = jnp.zeros((16,), jnp.float32)

        base_row = s * rows_per_sub
        is_last = s == NS - 1
        zoffs = [q * WIN for q in range(nfull)]
        if rem:
            zoffs.append(rows_per_sub - WIN)
        for off in zoffs:
            pltpu.sync_copy(rows0, acc_sh.at[pl.ds(base_row + off, WIN)])
        if tail_rows:
            @pl.when(is_last)
            def _():
                pltpu.sync_copy(rows0, acc_sh.at[pl.ds(n - WIN, WIN)])

        if count_only:
            @pl.loop(0, WIN)
            def _(r):
                @pl.loop(0, d, step=16)
                def _(q):
                    rows0[r, pl.ds(q, 16)] = jnp.ones((16,), jnp.float32)

        plsc.subcore_barrier()

        def run_chunk(m, wbase):
            pltpu.sync_copy(ei_hbm.at[1, pl.ds(wbase, m)],
                            dst_all.at[pl.ds(0, m)])
            if count_only:
                @pl.loop(0, m)
                def _(t):
                    pltpu.sync_copy(rows0, acc_sh.at[dst_all.at[t]], add=True)
                return
            pltpu.sync_copy(ei_hbm.at[0, pl.ds(wbase, m)],
                            src_all.at[pl.ds(0, m)])
            pltpu.async_copy(h_hbm.at[src_all.at[0]], rows0, sem0)
            pltpu.async_copy(h_hbm.at[src_all.at[1]], rows1, sem1)

            @pl.loop(0, m // 2)
            def _(u):
                t0 = 2 * u
                pltpu.make_async_copy(h_hbm.at[src_all.at[t0]], rows0,
                                      sem0).wait()
                pltpu.sync_copy(rows0, acc_sh.at[dst_all.at[t0]], add=True)

                @pl.when(t0 + 2 < m)
                def _():
                    pltpu.async_copy(h_hbm.at[src_all.at[t0 + 2]], rows0, sem0)

                pltpu.make_async_copy(h_hbm.at[src_all.at[t0 + 1]], rows1,
                                      sem1).wait()
                pltpu.sync_copy(rows1, acc_sh.at[dst_all.at[t0 + 1]], add=True)

                @pl.when(t0 + 3 < m)
                def _():
                    pltpu.async_copy(h_hbm.at[src_all.at[t0 + 3]], rows1, sem1)

        @pl.when(k < NW - 1)
        def _():
            run_chunk(half, k * chunk)
            run_chunk(half, k * chunk + half)

        @pl.when(k == NW - 1)
        def _():
            run_chunk(tail_chunk, (NW - 1) * chunk)

        plsc.subcore_barrier()

        pltpu.sync_copy(acc_sh.at[pl.ds(base_row, rows_per_sub)],
                        out_hbm.at[c, pl.ds(base_row, rows_per_sub)])
        if tail_rows:
            @pl.when(is_last)
            def _():
                pltpu.sync_copy(acc_sh.at[pl.ds(n - tail_rows, tail_rows)],
                                out_hbm.at[c, pl.ds(n - tail_rows, tail_rows)])

    return pl.kernel(body, out_type=out_type, mesh=mesh, scratch_types=scratch)


def _dot(a, b):
    return lax.dot_general(a, b, (((1,), (0,)), ((), ())),
                           preferred_element_type=jnp.float32)


def _matmul_bias(x, w, b, blk=1000):
    n, k = x.shape
    m = w.shape[1]
    assert n % blk == 0

    def body(x_ref, w_ref, b_ref, o_ref):
        o_ref[...] = _dot(x_ref[...], w_ref[...]) + b_ref[...]

    return pl.pallas_call(
        body,
        grid=(n // blk,),
        in_specs=[
            pl.BlockSpec((blk, k), lambda i: (i, 0)),
            pl.BlockSpec((k, m), lambda i: (0, 0)),
            pl.BlockSpec((1, m), lambda i: (0, 0)),
        ],
        out_specs=pl.BlockSpec((blk, m), lambda i: (i, 0)),
        out_shape=jax.ShapeDtypeStruct((n, m), jnp.float32),
    )(x, w, b)


def _sage_combine(h_self_b, part, degp, w_neigh, w_out=None, b_out=None,
                  blk=1000):
    n, d = h_self_b.shape
    fuse = w_out is not None
    m = w_out.shape[1] if fuse else d
    assert n % blk == 0

    def body(s_ref, p_ref, dg_ref, wn_ref, *rest):
        if fuse:
            wo_ref, bo_ref, o_ref = rest
        else:
            (o_ref,) = rest
        agg = p_ref[0] + p_ref[1]
        deg = dg_ref[0, :, 0:1] + dg_ref[1, :, 0:1]
        mean = agg / jnp.maximum(deg, 1.0)
        z = s_ref[...] + _dot(mean, wn_ref[...])
        z = jnp.where(z >= 0, z, NEG_SLOPE * z)
        if fuse:
            z = _dot(z, wo_ref[...]) + bo_ref[...]
        o_ref[...] = z

    in_specs = [
        pl.BlockSpec((blk, d), lambda i: (i, 0)),
        pl.BlockSpec((NC, blk, d), lambda i: (0, i, 0)),
        pl.BlockSpec((NC, blk, d), lambda i: (0, i, 0)),
        pl.BlockSpec((d, d), lambda i: (0, 0)),
    ]
    args = [h_self_b, part, degp, w_neigh]
    if fuse:
        in_specs.append(pl.BlockSpec((d, m), lambda i: (0, 0)))
        in_specs.append(pl.BlockSpec((1, m), lambda i: (0, 0)))
        args.extend([w_out, b_out])

    return pl.pallas_call(
        body,
        grid=(n // blk,),
        in_specs=in_specs,
        out_specs=pl.BlockSpec((blk, m), lambda i: (i, 0)),
        out_shape=jax.ShapeDtypeStruct((n, m), jnp.float32),
    )(*args)


def kernel(features, edge_index, W_in, b_in, W_self1, W_neigh1, b1,
           W_self2, W_neigh2, b2, W_out, b_out):
    n, d = features.shape
    e = edge_index.shape[1]
    ei = edge_index.reshape(2, e // WIN, WIN)

    h0 = _matmul_bias(features, W_in, b_in.reshape(1, -1))
    seg = _seg_sum_sc(n, d, e)
    degp = _seg_sum_sc(n, d, e, count_only=True)(features, ei)
    part1 = seg(h0, ei)
    self1 = _matmul_bias(h0, W_self1, b1.reshape(1, -1))
    h1 = _sage_combine(self1, part1, degp, W_neigh1)
    part2 = seg(h1, ei)
    self2 = _matmul_bias(h1, W_self2, b2.reshape(1, -1))
    out = _sage_combine(self2, part2, degp, W_neigh2,
                        W_out, b_out.reshape(1, -1))
    return out


# speedup vs baseline: 12.0581x; 1.1250x over previous
import jax
import jax.numpy as jnp
from jax import lax
from jax.experimental import pallas as pl
from jax.experimental.pallas import tpu as pltpu
from jax.experimental.pallas import tpu_sc as plsc

NEG_SLOPE = 0.01
NC = 2
NS = 16
WIN = 128


def _seg_sum_sc(n, d, e, count_only=False):
    nwin = e // WIN
    assert nwin * WIN == e
    NW = NC * NS
    chunk = (-(-nwin // NW) + 7) // 8 * 8
    tail_chunk = nwin - chunk * (NW - 1)
    assert 0 < tail_chunk <= chunk and tail_chunk % 2 == 0 and chunk % 2 == 0
    rows_per_sub = (n // NS) // 8 * 8
    tail_rows = n - rows_per_sub * NS
    assert rows_per_sub % 8 == 0 and tail_rows % 8 == 0
    nfull = rows_per_sub // WIN
    rem = rows_per_sub % WIN

    da = 32 if count_only else d
    mesh = plsc.VectorSubcoreMesh(core_axis_name="c", subcore_axis_name="s")
    out_type = jax.ShapeDtypeStruct((NC, n, da), jnp.float32)
    half = chunk // 2
    assert half % 2 == 0 and tail_chunk <= half
    scratch = [
        pltpu.VMEM((half, WIN), jnp.int32),
        pltpu.VMEM((half, WIN), jnp.int32),
        pltpu.VMEM((WIN, da), jnp.float32),
        pltpu.VMEM((WIN, d), jnp.float32),
        pltpu.VMEM_SHARED((n, da), jnp.float32),
        pltpu.SemaphoreType.DMA,
        pltpu.SemaphoreType.DMA,
    ]

    def body(h_hbm, ei_hbm, out_hbm, src_all, dst_all, rows0, rows1,
             acc_sh, sem0, sem1):
        c = lax.axis_index("c")
        s = lax.axis_index("s")
        k = c * NS + s

        @pl.loop(0, WIN)
        def _(r):
            @pl.loop(0, da, step=16)
            def _(q):
                rows0[r, pl.ds(q, 16)] = jnp.zeros((16,), jnp.float32)

        base_row = s * rows_per_sub
        is_last = s == NS - 1
        zoffs = [q * WIN for q in range(nfull)]
        if rem:
            zoffs.append(rows_per_sub - WIN)
        for off in zoffs:
            pltpu.sync_copy(rows0, acc_sh.at[pl.ds(base_row + off, WIN)])
        if tail_rows:
            @pl.when(is_last)
            def _():
                pltpu.sync_copy(rows0, acc_sh.at[pl.ds(n - WIN, WIN)])

        if count_only:
            @pl.loop(0, WIN)
            def _(r):
                @pl.loop(0, da, step=16)
                def _(q):
                    rows0[r, pl.ds(q, 16)] = jnp.ones((16,), jnp.float32)

        plsc.subcore_barrier()

        def run_chunk(m, wbase):
            pltpu.sync_copy(ei_hbm.at[1, pl.ds(wbase, m)],
                            dst_all.at[pl.ds(0, m)])
            if count_only:
                @pl.loop(0, m)
                def _(t):
                    pltpu.sync_copy(rows0, acc_sh.at[dst_all.at[t]], add=True)
                return
            pltpu.sync_copy(ei_hbm.at[0, pl.ds(wbase, m)],
                            src_all.at[pl.ds(0, m)])
            pltpu.async_copy(h_hbm.at[src_all.at[0]], rows0, sem0)
            pltpu.async_copy(h_hbm.at[src_all.at[1]], rows1, sem1)

            @pl.loop(0, m // 2)
            def _(u):
                t0 = 2 * u
                pltpu.make_async_copy(h_hbm.at[src_all.at[t0]], rows0,
                                      sem0).wait()
                pltpu.sync_copy(rows0, acc_sh.at[dst_all.at[t0]], add=True)

                @pl.when(t0 + 2 < m)
                def _():
                    pltpu.async_copy(h_hbm.at[src_all.at[t0 + 2]], rows0, sem0)

                pltpu.make_async_copy(h_hbm.at[src_all.at[t0 + 1]], rows1,
                                      sem1).wait()
                pltpu.sync_copy(rows1, acc_sh.at[dst_all.at[t0 + 1]], add=True)

                @pl.when(t0 + 3 < m)
                def _():
                    pltpu.async_copy(h_hbm.at[src_all.at[t0 + 3]], rows1, sem1)

        @pl.when(k < NW - 1)
        def _():
            run_chunk(half, k * chunk)
            run_chunk(half, k * chunk + half)

        @pl.when(k == NW - 1)
        def _():
            run_chunk(tail_chunk, (NW - 1) * chunk)

        plsc.subcore_barrier()

        pltpu.sync_copy(acc_sh.at[pl.ds(base_row, rows_per_sub)],
                        out_hbm.at[c, pl.ds(base_row, rows_per_sub)])
        if tail_rows:
            @pl.when(is_last)
            def _():
                pltpu.sync_copy(acc_sh.at[pl.ds(n - tail_rows, tail_rows)],
                                out_hbm.at[c, pl.ds(n - tail_rows, tail_rows)])

    return pl.kernel(body, out_type=out_type, mesh=mesh, scratch_types=scratch)


def _dot(a, b):
    return lax.dot_general(a, b, (((1,), (0,)), ((), ())),
                           preferred_element_type=jnp.float32)


def _matmul_bias(x, w, b, blk=1000):
    n, k = x.shape
    m = w.shape[1]
    assert n % blk == 0

    def body(x_ref, w_ref, b_ref, o_ref):
        o_ref[...] = _dot(x_ref[...], w_ref[...]) + b_ref[...]

    return pl.pallas_call(
        body,
        grid=(n // blk,),
        in_specs=[
            pl.BlockSpec((blk, k), lambda i: (i, 0)),
            pl.BlockSpec((k, m), lambda i: (0, 0)),
            pl.BlockSpec((1, m), lambda i: (0, 0)),
        ],
        out_specs=pl.BlockSpec((blk, m), lambda i: (i, 0)),
        out_shape=jax.ShapeDtypeStruct((n, m), jnp.float32),
    )(x, w, b)


def _sage_combine(h_self_b, part, degp, w_neigh, w_out=None, b_out=None,
                  blk=1000):
    n, d = h_self_b.shape
    fuse = w_out is not None
    m = w_out.shape[1] if fuse else d
    assert n % blk == 0

    def body(s_ref, p_ref, dg_ref, wn_ref, *rest):
        if fuse:
            wo_ref, bo_ref, o_ref = rest
        else:
            (o_ref,) = rest
        agg = p_ref[0] + p_ref[1]
        deg = dg_ref[0, :, 0:1] + dg_ref[1, :, 0:1]
        mean = agg / jnp.maximum(deg, 1.0)
        z = s_ref[...] + _dot(mean, wn_ref[...])
        z = jnp.where(z >= 0, z, NEG_SLOPE * z)
        if fuse:
            z = _dot(z, wo_ref[...]) + bo_ref[...]
        o_ref[...] = z

    in_specs = [
        pl.BlockSpec((blk, d), lambda i: (i, 0)),
        pl.BlockSpec((NC, blk, d), lambda i: (0, i, 0)),
        pl.BlockSpec((NC, blk, degp.shape[2]), lambda i: (0, i, 0)),
        pl.BlockSpec((d, d), lambda i: (0, 0)),
    ]
    args = [h_self_b, part, degp, w_neigh]
    if fuse:
        in_specs.append(pl.BlockSpec((d, m), lambda i: (0, 0)))
        in_specs.append(pl.BlockSpec((1, m), lambda i: (0, 0)))
        args.extend([w_out, b_out])

    return pl.pallas_call(
        body,
        grid=(n // blk,),
        in_specs=in_specs,
        out_specs=pl.BlockSpec((blk, m), lambda i: (i, 0)),
        out_shape=jax.ShapeDtypeStruct((n, m), jnp.float32),
    )(*args)


def kernel(features, edge_index, W_in, b_in, W_self1, W_neigh1, b1,
           W_self2, W_neigh2, b2, W_out, b_out):
    n, d = features.shape
    e = edge_index.shape[1]
    ei = edge_index.reshape(2, e // WIN, WIN)

    h0 = _matmul_bias(features, W_in, b_in.reshape(1, -1))
    seg = _seg_sum_sc(n, d, e)
    degp = _seg_sum_sc(n, d, e, count_only=True)(features, ei)
    part1 = seg(h0, ei)
    self1 = _matmul_bias(h0, W_self1, b1.reshape(1, -1))
    h1 = _sage_combine(self1, part1, degp, W_neigh1)
    part2 = seg(h1, ei)
    self2 = _matmul_bias(h1, W_self2, b2.reshape(1, -1))
    out = _sage_combine(self2, part2, degp, W_neigh2,
                        W_out, b_out.reshape(1, -1))
    return out
